# baseline (speedup 1.0000x reference)
import functools

import jax
import jax.numpy as jnp
import numpy as np
from jax import lax
from jax.experimental import pallas as pl
from jax.experimental.pallas import tpu as pltpu
from jax.experimental.pallas import tpu_sc as plsc

_B, _N, _CIN = 8, 4096, 32
_S = 1024
_RAD = 0.4
_NS = 32
_D = 128
_NC, _NSUB, _L = 2, 16, 16
_NW = _NC * _NSUB
_SEG = _S * _B // _NW
_CBLK = 128


def _fps_kernel(xyzT_ref, out_ref):
    arr = xyzT_ref[...]
    xyz24 = arr.reshape(3 * _B, _N)
    x = xyz24[0:_B]
    y = xyz24[_B:2 * _B]
    z = xyz24[2 * _B:3 * _B]
    iota = lax.broadcasted_iota(jnp.int32, (_B, _N), 1)
    jlane = lax.broadcasted_iota(jnp.int32, (1, 24), 1)
    isub = lax.broadcasted_iota(jnp.int32, (_B, 24), 0)
    eyef = (jlane % _B == isub).astype(jnp.float32)

    def step(t, carry):
        dists, far = carry
        sel = iota == far
        self = sel.astype(jnp.float32)
        mm = lax.dot_general(self, xyz24, (((1,), (1,)), ((), ())),
                             preferred_element_type=jnp.float32,
                             precision=lax.Precision.HIGHEST)
        row = jnp.sum(mm * eyef, axis=0, keepdims=True)
        out_ref[pl.ds(t, 1), :] = row
        eye8 = eyef[:, 0:_B]
        cx = jnp.sum(mm[:, 0:_B] * eye8, axis=1, keepdims=True)
        cy = jnp.sum(mm[:, _B:2 * _B] * eye8, axis=1, keepdims=True)
        cz = jnp.sum(mm[:, 2 * _B:3 * _B] * eye8, axis=1, keepdims=True)
        dx = x - cx
        dy = y - cy
        dz = z - cz
        d = dx * dx + dy * dy + dz * dz
        dists = jnp.minimum(dists, d)
        m = jnp.max(dists, axis=1, keepdims=True)
        far = jnp.min(jnp.where(dists == m, iota, _N), axis=1, keepdims=True)
        return dists, far.astype(jnp.int32)

    init = (jnp.full((_B, _N), 1e10, jnp.float32), jnp.zeros((_B, 1), jnp.int32))
    lax.fori_loop(0, _S, step, init)


_fps_call = pl.pallas_call(
    _fps_kernel,
    out_shape=jax.ShapeDtypeStruct((_S, 24), jnp.float32),
)


_R2 = np.float32(_RAD * _RAD)


def _bf16_round(v):
    u = plsc.bitcast(v, jnp.uint32)
    r = (u + jnp.uint32(0x7FFF) + ((u >> jnp.uint32(16)) & jnp.uint32(1)))
    return plsc.bitcast(r & jnp.uint32(0xFFFF0000), jnp.float32)


@functools.lru_cache(maxsize=None)
def _make_sc_group():
  mesh = plsc.VectorSubcoreMesh(core_axis_name="c", subcore_axis_name="s",
                                num_cores=_NC, num_subcores=_NSUB)

  @functools.partial(
    pl.kernel,
    out_type=jax.ShapeDtypeStruct((_B * _S * _NS, _D), jnp.float32),
    mesh=mesh,
    scratch_types=[
        pltpu.VMEM((_N,), jnp.float32),
        pltpu.VMEM((_N,), jnp.float32),
        pltpu.VMEM((_N,), jnp.float32),
        pltpu.VMEM((_N,), jnp.float32),
        pltpu.VMEM((_N,), jnp.float32),
        pltpu.VMEM((_N,), jnp.float32),
        pltpu.VMEM((_N,), jnp.float32),
        pltpu.VMEM((_SEG,), jnp.float32),
        pltpu.VMEM((_SEG,), jnp.float32),
        pltpu.VMEM((_SEG,), jnp.float32),
        pltpu.VMEM((_SEG * _NS,), jnp.int32),
        pltpu.VMEM((128, _D), jnp.float32),
        pltpu.VMEM((128, _D), jnp.float32),
        pltpu.SemaphoreType.DMA,
        pltpu.SemaphoreType.DMA,
    ],
    compiler_params=pltpu.CompilerParams(needs_layout_passes=False,
                                         use_tc_tiling_on_sc=False),
  )
  def _sc_group(xf, yf, zf, cxf, cyf, czf,
                tbl, g_out, xv, yv, zv, xbv, ybv, zbv, p2v,
                cxv, cyv, czv, idxv, rows, rows2, sem, sem2):
    wid = lax.axis_index("s") * _NC + lax.axis_index("c")
    b = wid // (_NW // _B)
    seg = wid % (_NW // _B)
    cbase = b * _S + seg * _SEG
    gbase = b * _N
    pltpu.sync_copy(xf.at[pl.ds(b * _N, _N)], xv)
    pltpu.sync_copy(yf.at[pl.ds(b * _N, _N)], yv)
    pltpu.sync_copy(zf.at[pl.ds(b * _N, _N)], zv)
    pltpu.sync_copy(cxf.at[pl.ds(cbase, _SEG)], cxv)
    pltpu.sync_copy(cyf.at[pl.ds(cbase, _SEG)], cyv)
    pltpu.sync_copy(czf.at[pl.ds(cbase, _SEG)], czv)

    iota16 = lax.iota(jnp.int32, 16)

    def p2_body(j, _):
        xs = xv[pl.ds(j * 16, 16)]
        ys = yv[pl.ds(j * 16, 16)]
        zs = zv[pl.ds(j * 16, 16)]
        p2v[pl.ds(j * 16, 16)] = (xs * xs + ys * ys) + zs * zs
        xbv[pl.ds(j * 16, 16)] = _bf16_round(xs)
        ybv[pl.ds(j * 16, 16)] = _bf16_round(ys)
        zbv[pl.ds(j * 16, 16)] = _bf16_round(zs)
        return 0

    lax.fori_loop(0, _N // 16, p2_body, 0)

    def center_body(s, _):
        sidx = jnp.full((16,), s, jnp.int32)
        cxs = plsc.load_gather(cxv, [sidx])
        cys = plsc.load_gather(cyv, [sidx])
        czs = plsc.load_gather(czv, [sidx])
        cxb = _bf16_round(cxs)
        cyb = _bf16_round(cys)
        czb = _bf16_round(czs)
        c2s = (cxs * cxs + cys * cys) + czs * czs

        def cond(carry):
            j, cnt = carry
            return jnp.logical_and(cnt < _NS, j < _N // 64)

        def body(carry):
            j, cnt = carry
            base = j * 64
            ms = []
            pcs = []
            for k in range(4):
                xs = xbv[pl.ds(base + k * 16, 16)]
                ys = ybv[pl.ds(base + k * 16, 16)]
                zs = zbv[pl.ds(base + k * 16, 16)]
                p2 = p2v[pl.ds(base + k * 16, 16)]
                dot = xs * cxb + ys * cyb + zs * czb
                sq = (c2s + p2) - 2.0 * dot
                m = sq <= _R2
                ms.append(m)
                pcs.append(plsc.all_reduce_population_count(m)[0])

            total = (pcs[0] + pcs[1]) + (pcs[2] + pcs[3])

            @pl.when(total > 0)
            def _():
                c = cnt
                for k in range(4):
                    pos = plsc.cumsum(ms[k].astype(jnp.int32))
                    slot = (s * _NS + c - 1) + pos
                    wm = jnp.logical_and(ms[k], (c + pos) <= _NS)
                    ivec = (gbase + base + k * 16) + iota16
                    plsc.store_scatter(idxv, [slot], ivec, mask=wm)
                    c = c + pcs[k]

            return j + 1, cnt + total

        _, cfin = lax.while_loop(cond, body, (jnp.int32(0), jnp.int32(0)))

        first = plsc.load_gather(idxv, [jnp.full((16,), s * _NS, jnp.int32)])
        first = jnp.where(jnp.full((16,), cfin, jnp.int32) > 0, first,
                          jnp.full((16,), gbase, jnp.int32))
        for h in range(_NS // 16):
            lanes = h * 16 + iota16
            plsc.store_scatter(idxv, [s * _NS + lanes], first,
                               mask=lanes >= cfin)
        return 0

    lax.fori_loop(0, _SEG, center_body, 0)

    nchunk = _SEG * _NS // 128

    def _start(i, buf, s):
        pltpu.async_copy(tbl.at[idxv.at[pl.ds(i * 128, 128)]], buf, s)

    _start(0, rows, sem)

    def gather_body(t, _):
        i = t * 2
        _start(i + 1, rows2, sem2)
        pltpu.make_async_copy(tbl.at[idxv.at[pl.ds(i * 128, 128)]], rows,
                              sem).wait()
        pltpu.sync_copy(rows, g_out.at[pl.ds(cbase * _NS + i * 128, 128)])

        @pl.when(t + 1 < nchunk // 2)
        def _():
            _start(i + 2, rows, sem)

        pltpu.make_async_copy(tbl.at[idxv.at[pl.ds((i + 1) * 128, 128)]],
                              rows2, sem2).wait()
        pltpu.sync_copy(rows2, g_out.at[pl.ds(cbase * _NS + (i + 1) * 128, 128)])
        return 0

    lax.fori_loop(0, nchunk // 2, gather_body, 0)

  return _sc_group


def _mlp_kernel(g_ref, nx_ref, w1_ref, b1_ref, w2_ref, b2_ref, w3_ref, b3_ref,
                out_ref):
    g = g_ref[...]
    nx = nx_ref[...]
    w1 = w1_ref[...]
    bc = b1_ref[...] - lax.dot_general(nx, w1[0:3, :], (((1,), (0,)), ((), ())),
                                       preferred_element_type=jnp.float32)
    bcr = jnp.broadcast_to(bc[:, None, :], (_CBLK, _NS, 32)).reshape(
        _CBLK * _NS, 32)
    h = jnp.maximum(jnp.dot(g, w1, preferred_element_type=jnp.float32) + bcr, 0.0)
    h = jnp.maximum(jnp.dot(h, w2_ref[...], preferred_element_type=jnp.float32)
                    + b2_ref[...], 0.0)
    h = jnp.maximum(jnp.dot(h, w3_ref[...], preferred_element_type=jnp.float32)
                    + b3_ref[...], 0.0)
    pooled = jnp.max(h.reshape(_CBLK, _NS, 64), axis=1)
    out_ref[...] = pooled.T[None]


_mlp_call = pl.pallas_call(
    _mlp_kernel,
    grid=(_B * _S // _CBLK,),
    in_specs=[
        pl.BlockSpec((_CBLK * _NS, _D), lambda i: (i, 0)),
        pl.BlockSpec((_CBLK, 3), lambda i: (i, 0)),
        pl.BlockSpec((_D, 32), lambda i: (0, 0)),
        pl.BlockSpec((1, 32), lambda i: (0, 0)),
        pl.BlockSpec((32, 32), lambda i: (0, 0)),
        pl.BlockSpec((1, 32), lambda i: (0, 0)),
        pl.BlockSpec((32, 64), lambda i: (0, 0)),
        pl.BlockSpec((1, 64), lambda i: (0, 0)),
    ],
    out_specs=pl.BlockSpec((1, 64, _CBLK),
                           lambda i: (i // (_S // _CBLK), 0, i % (_S // _CBLK))),
    out_shape=jax.ShapeDtypeStruct((_B, 64, _S), jnp.float32),
)


def kernel(xyz, features, W1, b1, W2, b2, W3, b3):
    xyzT = jnp.transpose(xyz, (2, 0, 1))
    fps = _fps_call(xyzT)
    cx = fps[:, 0:_B].T
    cy = fps[:, _B:2 * _B].T
    cz = fps[:, 2 * _B:3 * _B].T
    new_xyz = jnp.stack([cx, cy, cz], axis=-1)

    tbl = jnp.concatenate(
        [xyz, features, jnp.zeros((_B, _N, _D - 3 - _CIN), jnp.float32)],
        axis=-1).reshape(_B * _N, _D)
    g = _make_sc_group()(xyzT[0].reshape(-1), xyzT[1].reshape(-1),
                         xyzT[2].reshape(-1),
                         cx.reshape(-1), cy.reshape(-1), cz.reshape(-1),
                         tbl)

    W1p = jnp.concatenate([W1, jnp.zeros((_D - 35, 32), jnp.float32)], axis=0)
    new_features = _mlp_call(g, new_xyz.reshape(_B * _S, 3), W1p,
                             b1.reshape(1, 32), W2, b2.reshape(1, 32),
                             W3, b3.reshape(1, 64))
    return new_xyz, new_features


# speedup vs baseline: 1.2654x; 1.2654x over previous
import functools

import jax
import jax.numpy as jnp
import numpy as np
from jax import lax
from jax.experimental import pallas as pl
from jax.experimental.pallas import tpu as pltpu
from jax.experimental.pallas import tpu_sc as plsc

_B, _N, _CIN = 8, 4096, 32
_S = 1024
_RAD = 0.4
_NS = 32
_D = 128
_NC, _NSUB, _L = 2, 16, 16
_NW = _NC * _NSUB
_SEG = _S * _B // _NW
_CBLK = 128


def _fps_kernel(xyzT_ref, out_ref):
    arr = xyzT_ref[...]
    xyz24 = arr.reshape(3 * _B, _N)
    x = xyz24[0:_B]
    y = xyz24[_B:2 * _B]
    z = xyz24[2 * _B:3 * _B]
    iota = lax.broadcasted_iota(jnp.int32, (_B, _N), 1)
    jlane = lax.broadcasted_iota(jnp.int32, (1, 24), 1)
    isub = lax.broadcasted_iota(jnp.int32, (_B, 24), 0)
    eyef = (jlane % _B == isub).astype(jnp.float32)

    def step(t, carry):
        dists, far = carry
        sel = iota == far
        self = sel.astype(jnp.float32)
        mm = lax.dot_general(self, xyz24, (((1,), (1,)), ((), ())),
                             preferred_element_type=jnp.float32,
                             precision=lax.Precision.HIGHEST)
        row = jnp.sum(mm * eyef, axis=0, keepdims=True)
        out_ref[pl.ds(t, 1), :] = row
        cx = jnp.sum(jnp.where(sel, x, 0.0), axis=1, keepdims=True)
        cy = jnp.sum(jnp.where(sel, y, 0.0), axis=1, keepdims=True)
        cz = jnp.sum(jnp.where(sel, z, 0.0), axis=1, keepdims=True)
        dx = x - cx
        dy = y - cy
        dz = z - cz
        d = dx * dx + dy * dy + dz * dz
        dists = jnp.minimum(dists, d)
        m = jnp.max(dists, axis=1, keepdims=True)
        far = jnp.min(jnp.where(dists == m, iota, _N), axis=1, keepdims=True)
        return dists, far.astype(jnp.int32)

    init = (jnp.full((_B, _N), 1e10, jnp.float32), jnp.zeros((_B, 1), jnp.int32))
    lax.fori_loop(0, _S, step, init)


_fps_call = pl.pallas_call(
    _fps_kernel,
    out_shape=jax.ShapeDtypeStruct((_S, 24), jnp.float32),
)


_R2 = np.float32(_RAD * _RAD)


def _bf16_round(v):
    u = plsc.bitcast(v, jnp.uint32)
    r = (u + jnp.uint32(0x7FFF) + ((u >> jnp.uint32(16)) & jnp.uint32(1)))
    return plsc.bitcast(r & jnp.uint32(0xFFFF0000), jnp.float32)


@functools.lru_cache(maxsize=None)
def _make_sc_group():
  mesh = plsc.VectorSubcoreMesh(core_axis_name="c", subcore_axis_name="s",
                                num_cores=_NC, num_subcores=_NSUB)

  @functools.partial(
    pl.kernel,
    out_type=jax.ShapeDtypeStruct((_B * _S * _NS, _D), jnp.float32),
    mesh=mesh,
    scratch_types=[
        pltpu.VMEM((_N,), jnp.float32),
        pltpu.VMEM((_N,), jnp.float32),
        pltpu.VMEM((_N,), jnp.float32),
        pltpu.VMEM((_N,), jnp.float32),
        pltpu.VMEM((_N,), jnp.float32),
        pltpu.VMEM((_N,), jnp.float32),
        pltpu.VMEM((_N,), jnp.float32),
        pltpu.VMEM((_SEG,), jnp.float32),
        pltpu.VMEM((_SEG,), jnp.float32),
        pltpu.VMEM((_SEG,), jnp.float32),
        pltpu.VMEM((_SEG * _NS,), jnp.int32),
        pltpu.VMEM((128, _D), jnp.float32),
        pltpu.VMEM((128, _D), jnp.float32),
        pltpu.SemaphoreType.DMA,
        pltpu.SemaphoreType.DMA,
    ],
    compiler_params=pltpu.CompilerParams(needs_layout_passes=False,
                                         use_tc_tiling_on_sc=False),
  )
  def _sc_group(xf, yf, zf, cxf, cyf, czf,
                tbl, g_out, xv, yv, zv, xbv, ybv, zbv, p2v,
                cxv, cyv, czv, idxv, rows, rows2, sem, sem2):
    wid = lax.axis_index("s") * _NC + lax.axis_index("c")
    b = wid // (_NW // _B)
    seg = wid % (_NW // _B)
    cbase = b * _S + seg * _SEG
    gbase = b * _N
    pltpu.sync_copy(xf.at[pl.ds(b * _N, _N)], xv)
    pltpu.sync_copy(yf.at[pl.ds(b * _N, _N)], yv)
    pltpu.sync_copy(zf.at[pl.ds(b * _N, _N)], zv)
    pltpu.sync_copy(cxf.at[pl.ds(cbase, _SEG)], cxv)
    pltpu.sync_copy(cyf.at[pl.ds(cbase, _SEG)], cyv)
    pltpu.sync_copy(czf.at[pl.ds(cbase, _SEG)], czv)

    iota16 = lax.iota(jnp.int32, 16)

    def p2_body(j, _):
        xs = xv[pl.ds(j * 16, 16)]
        ys = yv[pl.ds(j * 16, 16)]
        zs = zv[pl.ds(j * 16, 16)]
        p2v[pl.ds(j * 16, 16)] = (xs * xs + ys * ys) + zs * zs
        xbv[pl.ds(j * 16, 16)] = _bf16_round(xs)
        ybv[pl.ds(j * 16, 16)] = _bf16_round(ys)
        zbv[pl.ds(j * 16, 16)] = _bf16_round(zs)
        return 0

    lax.fori_loop(0, _N // 16, p2_body, 0)

    def center_body(s, _):
        sidx = jnp.full((16,), s, jnp.int32)
        cxs = plsc.load_gather(cxv, [sidx])
        cys = plsc.load_gather(cyv, [sidx])
        czs = plsc.load_gather(czv, [sidx])
        cxb = _bf16_round(cxs)
        cyb = _bf16_round(cys)
        czb = _bf16_round(czs)
        c2s = (cxs * cxs + cys * cys) + czs * czs

        def cond(carry):
            j, cnt = carry
            return jnp.logical_and(cnt < _NS, j < _N // 64)

        def body(carry):
            j, cnt = carry
            base = j * 64
            ms = []
            pcs = []
            for k in range(4):
                xs = xbv[pl.ds(base + k * 16, 16)]
                ys = ybv[pl.ds(base + k * 16, 16)]
                zs = zbv[pl.ds(base + k * 16, 16)]
                p2 = p2v[pl.ds(base + k * 16, 16)]
                dot = xs * cxb + ys * cyb + zs * czb
                sq = (c2s + p2) - 2.0 * dot
                m = sq <= _R2
                ms.append(m)
                pcs.append(plsc.all_reduce_population_count(m)[0])

            total = (pcs[0] + pcs[1]) + (pcs[2] + pcs[3])

            @pl.when(total > 0)
            def _():
                c = cnt
                for k in range(4):
                    pos = plsc.cumsum(ms[k].astype(jnp.int32))
                    slot = (s * _NS + c - 1) + pos
                    wm = jnp.logical_and(ms[k], (c + pos) <= _NS)
                    ivec = (gbase + base + k * 16) + iota16
                    plsc.store_scatter(idxv, [slot], ivec, mask=wm)
                    c = c + pcs[k]

            return j + 1, cnt + total

        _, cfin = lax.while_loop(cond, body, (jnp.int32(0), jnp.int32(0)))

        first = plsc.load_gather(idxv, [jnp.full((16,), s * _NS, jnp.int32)])
        first = jnp.where(jnp.full((16,), cfin, jnp.int32) > 0, first,
                          jnp.full((16,), gbase, jnp.int32))
        for h in range(_NS // 16):
            lanes = h * 16 + iota16
            plsc.store_scatter(idxv, [s * _NS + lanes], first,
                               mask=lanes >= cfin)
        return 0

    lax.fori_loop(0, _SEG, center_body, 0)

    nchunk = _SEG * _NS // 128

    def _start(i, buf, s):
        pltpu.async_copy(tbl.at[idxv.at[pl.ds(i * 128, 128)]], buf, s)

    _start(0, rows, sem)

    def gather_body(t, _):
        i = t * 2
        _start(i + 1, rows2, sem2)
        pltpu.make_async_copy(tbl.at[idxv.at[pl.ds(i * 128, 128)]], rows,
                              sem).wait()
        pltpu.sync_copy(rows, g_out.at[pl.ds(cbase * _NS + i * 128, 128)])

        @pl.when(t + 1 < nchunk // 2)
        def _():
            _start(i + 2, rows, sem)

        pltpu.make_async_copy(tbl.at[idxv.at[pl.ds((i + 1) * 128, 128)]],
                              rows2, sem2).wait()
        pltpu.sync_copy(rows2, g_out.at[pl.ds(cbase * _NS + (i + 1) * 128, 128)])
        return 0

    lax.fori_loop(0, nchunk // 2, gather_body, 0)

  return _sc_group


def _mlp_kernel(g_ref, nx_ref, w1_ref, b1_ref, w2_ref, b2_ref, w3_ref, b3_ref,
                out_ref):
    g = g_ref[...]
    nx = nx_ref[...]
    w1 = w1_ref[...]
    bc = b1_ref[...] - lax.dot_general(nx, w1[0:3, :], (((1,), (0,)), ((), ())),
                                       preferred_element_type=jnp.float32)
    bcr = jnp.broadcast_to(bc[:, None, :], (_CBLK, _NS, 32)).reshape(
        _CBLK * _NS, 32)
    h = jnp.maximum(jnp.dot(g, w1, preferred_element_type=jnp.float32) + bcr, 0.0)
    h = jnp.maximum(jnp.dot(h, w2_ref[...], preferred_element_type=jnp.float32)
                    + b2_ref[...], 0.0)
    h = jnp.maximum(jnp.dot(h, w3_ref[...], preferred_element_type=jnp.float32)
                    + b3_ref[...], 0.0)
    pooled = jnp.max(h.reshape(_CBLK, _NS, 64), axis=1)
    out_ref[...] = pooled.T[None]


_mlp_call = pl.pallas_call(
    _mlp_kernel,
    grid=(_B * _S // _CBLK,),
    in_specs=[
        pl.BlockSpec((_CBLK * _NS, _D), lambda i: (i, 0)),
        pl.BlockSpec((_CBLK, 3), lambda i: (i, 0)),
        pl.BlockSpec((_D, 32), lambda i: (0, 0)),
        pl.BlockSpec((1, 32), lambda i: (0, 0)),
        pl.BlockSpec((32, 32), lambda i: (0, 0)),
        pl.BlockSpec((1, 32), lambda i: (0, 0)),
        pl.BlockSpec((32, 64), lambda i: (0, 0)),
        pl.BlockSpec((1, 64), lambda i: (0, 0)),
    ],
    out_specs=pl.BlockSpec((1, 64, _CBLK),
                           lambda i: (i // (_S // _CBLK), 0, i % (_S // _CBLK))),
    out_shape=jax.ShapeDtypeStruct((_B, 64, _S), jnp.float32),
)


def kernel(xyz, features, W1, b1, W2, b2, W3, b3):
    xyzT = jnp.transpose(xyz, (2, 0, 1))
    fps = _fps_call(xyzT)
    cx = fps[:, 0:_B].T
    cy = fps[:, _B:2 * _B].T
    cz = fps[:, 2 * _B:3 * _B].T
    new_xyz = jnp.stack([cx, cy, cz], axis=-1)

    tbl = jnp.concatenate(
        [xyz, features, jnp.zeros((_B, _N, _D - 3 - _CIN), jnp.float32)],
        axis=-1).reshape(_B * _N, _D)
    g = _make_sc_group()(xyzT[0].reshape(-1), xyzT[1].reshape(-1),
                         xyzT[2].reshape(-1),
                         cx.reshape(-1), cy.reshape(-1), cz.reshape(-1),
                         tbl)

    W1p = jnp.concatenate([W1, jnp.zeros((_D - 35, 32), jnp.float32)], axis=0)
    new_features = _mlp_call(g, new_xyz.reshape(_B * _S, 3), W1p,
                             b1.reshape(1, 32), W2, b2.reshape(1, 32),
                             W3, b3.reshape(1, 64))
    return new_xyz, new_features
